# baseline (speedup 1.0000x reference)
import functools

import jax
import jax.numpy as jnp
from jax import lax
from jax.experimental import pallas as pl
from jax.experimental.pallas import tpu as pltpu
from jax.experimental.pallas import tpu_sc as plsc

N = 10000
E = 320000
IN_DIM = 128
HID = 64
LAT = 32
OUT_DIM = 128

NC = 2
NS = 16
NW = NC * NS
LANES = 16
CHUNK = 80
EPT = E // NW
NP = 10240
RPS = NP // NS


def _leaky(x):
    return jnp.where(x > 0, x, 0.2 * x)


def _pre1_body(x_ref, w1_ref, a1s_ref, a1d_ref,
               h_ref, as_ref, ad_ref, off_ref):
    h = jnp.dot(x_ref[...], w1_ref[...], preferred_element_type=jnp.float32)
    asv = jnp.dot(h, a1s_ref[...][:, None],
                  preferred_element_type=jnp.float32)[:, 0]
    adv = jnp.dot(h, a1d_ref[...][:, None],
                  preferred_element_type=jnp.float32)[:, 0]
    m = jnp.max(asv)
    off = _leaky(m + adv)
    h_ref[...] = h
    as_ref[...] = asv
    ad_ref[...] = adv
    off_ref[...] = off


def _mid_body(pa_ref, pb_ref, b1_ref, w2_ref, a2s_ref, a2d_ref,
              h_ref, as_ref, ad_ref, off_ref):
    acc = pa_ref[0, 0:N] + pa_ref[1, 0:N]
    den = pb_ref[0, 0:N] + pb_ref[1, 0:N]
    h1 = jnp.maximum(acc / (den[:, None] + 1e-16) + b1_ref[...], 0.0)
    h2 = jnp.dot(h1, w2_ref[...], preferred_element_type=jnp.float32)
    asv = jnp.dot(h2, a2s_ref[...][:, None],
                  preferred_element_type=jnp.float32)[:, 0]
    adv = jnp.dot(h2, a2d_ref[...][:, None],
                  preferred_element_type=jnp.float32)[:, 0]
    m = jnp.max(asv)
    off = _leaky(m + adv)
    h_ref[...] = h2
    as_ref[...] = asv
    ad_ref[...] = adv
    off_ref[...] = off


def _post_body(pa_ref, pb_ref, b2_ref, wmu_ref, bmu_ref, wlv_ref, blv_ref,
               wd1_ref, bd1_ref, wd2_ref, bd2_ref, eps_ref,
               recon_ref, mu_ref, lv_ref):
    acc = pa_ref[0, 0:N] + pa_ref[1, 0:N]
    den = pb_ref[0, 0:N] + pb_ref[1, 0:N]
    h2 = jnp.maximum(acc / (den[:, None] + 1e-16) + b2_ref[...], 0.0)
    mu = jnp.dot(h2, wmu_ref[...], preferred_element_type=jnp.float32) + bmu_ref[...]
    lv = jnp.dot(h2, wlv_ref[...], preferred_element_type=jnp.float32) + blv_ref[...]
    z = mu + eps_ref[...] * jnp.exp(0.5 * lv)
    d = jnp.maximum(
        jnp.dot(z, wd1_ref[...], preferred_element_type=jnp.float32) + bd1_ref[...],
        0.0)
    recon_ref[...] = jax.nn.sigmoid(
        jnp.dot(d, wd2_ref[...], preferred_element_type=jnp.float32) + bd2_ref[...])
    mu_ref[...] = mu
    lv_ref[...] = lv


def _edge_sc(d, h, asv, adv, off, src, dst):
    gp = d // LANES
    zrows = RPS // 5
    nsub = N // NS
    mesh = plsc.VectorSubcoreMesh(core_axis_name="c", subcore_axis_name="s")

    nchunks = EPT // CHUNK

    @functools.partial(
        pl.kernel,
        mesh=mesh,
        compiler_params=pltpu.CompilerParams(needs_layout_passes=False,
                                             use_tc_tiling_on_sc=False),
        out_type=[jax.ShapeDtypeStruct((NC, NP, d), jnp.float32),
                  jax.ShapeDtypeStruct((NC, NP, LANES), jnp.float32)],
        scratch_types=[
            pltpu.VMEM((N,), jnp.float32),
            pltpu.VMEM((N,), jnp.float32),
            pltpu.VMEM((N,), jnp.float32),
            pltpu.VMEM((nchunks, CHUNK), jnp.int32),
            pltpu.VMEM((nchunks, CHUNK), jnp.int32),
            pltpu.VMEM((CHUNK, d), jnp.float32),
            pltpu.VMEM((CHUNK, d), jnp.float32),
            pltpu.VMEM((CHUNK, LANES), jnp.float32),
            pltpu.VMEM((CHUNK, LANES), jnp.float32),
            pltpu.VMEM((zrows, d), jnp.float32),
            pltpu.VMEM((zrows, LANES), jnp.float32),
            pltpu.VMEM_SHARED((NP, d), jnp.float32),
            pltpu.VMEM_SHARED((NP, LANES), jnp.float32),
            pltpu.SemaphoreType.DMA,
            pltpu.SemaphoreType.DMA,
            pltpu.SemaphoreType.DMA,
            pltpu.SemaphoreType.DMA,
            pltpu.SemaphoreType.DMA,
            pltpu.SemaphoreType.DMA,
        ],
    )
    def k(h_hbm, as_hbm, ad_hbm, off_hbm, src_hbm, dst_hbm,
          partA_hbm, partB_hbm,
          as_t, ad_t, off_t, sidx, didx, rowsA0, rowsA1, rowsB0, rowsB1,
          zbufA, zbufB, numA_sh, numB_sh,
          gsem0, gsem1, sa0, sa1, sb0, sb1):
        c = lax.axis_index("c")
        s = lax.axis_index("s")
        wid = s * NC + c
        rowsA = (rowsA0, rowsA1)
        rowsB = (rowsB0, rowsB1)
        gsem = (gsem0, gsem1)
        sa = (sa0, sa1)
        sb = (sb0, sb1)

        pltpu.sync_copy(src_hbm.at[pl.ds(wid * nchunks, nchunks)], sidx)
        pltpu.sync_copy(dst_hbm.at[pl.ds(wid * nchunks, nchunks)], didx)
        pltpu.sync_copy(as_hbm, as_t)
        pltpu.sync_copy(ad_hbm, ad_t)
        pltpu.sync_copy(off_hbm, off_t)

        def _z(i, _):
            for g in range(gp):
                zbufA[i, pl.ds(g * LANES, LANES)] = jnp.zeros((LANES,),
                                                              jnp.float32)
            zbufB[i, pl.ds(0, LANES)] = jnp.zeros((LANES,), jnp.float32)
            return 0
        lax.fori_loop(0, zrows, _z, 0)
        for b in range(5):
            pltpu.sync_copy(zbufA,
                            numA_sh.at[pl.ds(s * RPS + b * zrows, zrows)])
            pltpu.sync_copy(zbufB,
                            numB_sh.at[pl.ds(s * RPS + b * zrows, zrows)])
        plsc.subcore_barrier()

        def scale(kk, b):
            ra = rowsA[b]
            rb = rowsB[b]
            for j in range(CHUNK // LANES):
                sv = sidx[kk, pl.ds(j * LANES, LANES)]
                dv = didx[kk, pl.ds(j * LANES, LANES)]
                a_s = plsc.load_gather(as_t, [sv])
                a_d = plsc.load_gather(ad_t, [dv])
                o_d = plsc.load_gather(off_t, [dv])
                e = a_s + a_d
                e = jnp.where(e > 0, e, 0.2 * e)
                wv = jnp.exp(e - o_d)
                for i in range(LANES):
                    w0 = wv[i]
                    r = j * LANES + i
                    rb[r, pl.ds(0, LANES)] = jnp.full((LANES,), w0,
                                                      jnp.float32)
                    for g in range(gp):
                        ra[r, pl.ds(g * LANES, LANES)] = (
                            ra[r, pl.ds(g * LANES, LANES)] * w0)

        def start_gather(kk, b):
            return pltpu.async_copy(h_hbm.at[sidx.at[kk]], rowsA[b], gsem[b])

        def wait_scatter(kk, b):
            pltpu.make_async_copy(rowsA[b], numA_sh.at[didx.at[kk]],
                                  sa[b]).wait()
            pltpu.make_async_copy(rowsB[b], numB_sh.at[didx.at[kk]],
                                  sb[b]).wait()

        def start_scatter(kk, b):
            pltpu.async_copy(rowsA[b], numA_sh.at[didx.at[kk]], sa[b],
                             add=True)
            pltpu.async_copy(rowsB[b], numB_sh.at[didx.at[kk]], sb[b],
                             add=True)

        start_gather(0, 0)

        def pair(p, _):
            for b in range(2):
                kk = 2 * p + b

                @pl.when(kk >= 1)
                def _():
                    wait_scatter(kk - 1, 1 - b)
                start_gather(kk + 1, 1 - b)
                pltpu.make_async_copy(h_hbm.at[sidx.at[kk]],
                                      rowsA[b], gsem[b]).wait()
                scale(kk, b)
                start_scatter(kk, b)
            return 0

        lax.fori_loop(0, (nchunks - 1) // 2, pair, 0)
        last = nchunks - 1
        wait_scatter(last - 1, 1 - (last % 2))
        pltpu.make_async_copy(h_hbm.at[sidx.at[last]],
                              rowsA[last % 2], gsem[last % 2]).wait()
        scale(last, last % 2)
        start_scatter(last, last % 2)
        wait_scatter(last, last % 2)

        plsc.subcore_barrier()
        pltpu.sync_copy(numA_sh.at[pl.ds(s * RPS, RPS)],
                        partA_hbm.at[c, pl.ds(s * RPS, RPS)])
        pltpu.sync_copy(numB_sh.at[pl.ds(s * RPS, RPS)],
                        partB_hbm.at[c, pl.ds(s * RPS, RPS)])

    src2 = src.reshape(NW * nchunks, CHUNK)
    dst2 = dst.reshape(NW * nchunks, CHUNK)
    return k(h, asv, adv, off, src2, dst2)


def kernel(x, edge_index, W1, a1_src, a1_dst, b1, W2, a2_src, a2_dst, b2,
           Wmu, bmu, Wlv, blv, Wd1, bd1, Wd2, bd2):
    src = edge_index[0]
    dst = edge_index[1]

    tc_params = pltpu.CompilerParams(vmem_limit_bytes=63 * 1024 * 1024)
    h1, as1, ad1, off1 = pl.pallas_call(
        _pre1_body,
        compiler_params=tc_params,
        out_shape=[
            jax.ShapeDtypeStruct((N, HID), jnp.float32),
            jax.ShapeDtypeStruct((N,), jnp.float32),
            jax.ShapeDtypeStruct((N,), jnp.float32),
            jax.ShapeDtypeStruct((N,), jnp.float32),
        ],
    )(x, W1, a1_src, a1_dst)

    pa1, pb1 = _edge_sc(HID, h1, as1, ad1, off1, src, dst)

    h2, as2, ad2, off2 = pl.pallas_call(
        _mid_body,
        compiler_params=tc_params,
        out_shape=[
            jax.ShapeDtypeStruct((N, LAT), jnp.float32),
            jax.ShapeDtypeStruct((N,), jnp.float32),
            jax.ShapeDtypeStruct((N,), jnp.float32),
            jax.ShapeDtypeStruct((N,), jnp.float32),
        ],
    )(pa1, pb1[:, :, 0], b1, W2, a2_src, a2_dst)

    pa2, pb2 = _edge_sc(LAT, h2, as2, ad2, off2, src, dst)

    eps = jax.random.normal(jax.random.key(42), (N, LAT), dtype=jnp.float32)
    recon, mu, logvar = pl.pallas_call(
        _post_body,
        compiler_params=tc_params,
        out_shape=[
            jax.ShapeDtypeStruct((N, OUT_DIM), jnp.float32),
            jax.ShapeDtypeStruct((N, LAT), jnp.float32),
            jax.ShapeDtypeStruct((N, LAT), jnp.float32),
        ],
    )(pa2, pb2[:, :, 0], b2, Wmu, bmu, Wlv, blv, Wd1, bd1, Wd2, bd2, eps)

    return (recon, mu, logvar)
